# baseline (speedup 1.0000x reference)
import jax
import jax.numpy as jnp
from jax.experimental import pallas as pl
from jax.experimental.pallas import tpu as pltpu


def _focal_body(x_ref, t_ref, o_ref, acc_ref):
    i = pl.program_id(0)
    g = pl.num_programs(0)

    @pl.when(i == 0)
    def _():
        acc_ref[0] = 0.0
        acc_ref[1] = 0.0

    x = x_ref[...]
    t = t_ref[...]
    e = jnp.exp(-jnp.abs(x))
    l1p = jnp.log1p(e)
    r = 1.0 / (1.0 + e)
    s = jnp.where(x >= 0.0, r, 1.0 - r)
    sp = jnp.maximum(x, 0.0) + l1p
    oms = 1.0 - s
    pos = 0.25 * oms * oms * (sp - x)
    neg = 0.75 * s * s * sp
    loss = jnp.where(t > 0.0, pos, neg)
    acc_ref[0] += jnp.sum(loss)
    acc_ref[1] += jnp.sum(jnp.max(t, axis=1))

    @pl.when(i == g - 1)
    def _():
        o_ref[0, 0] = acc_ref[0] / jnp.maximum(acc_ref[1], 1.0)


def kernel(logits, targets):
    n, c = logits.shape
    br = 1280
    grid = n // br
    out = pl.pallas_call(
        _focal_body,
        grid=(grid,),
        in_specs=[
            pl.BlockSpec((br, c), lambda i: (i, 0)),
            pl.BlockSpec((br, c), lambda i: (i, 0)),
        ],
        out_specs=pl.BlockSpec((1, 1), lambda i: (0, 0), memory_space=pltpu.SMEM),
        out_shape=jax.ShapeDtypeStruct((1, 1), jnp.float32),
        scratch_shapes=[pltpu.SMEM((2,), jnp.float32)],
        compiler_params=pltpu.CompilerParams(
            dimension_semantics=("arbitrary",),
        ),
    )(logits, targets)
    return out[0, 0]


# speedup vs baseline: 1.3197x; 1.3197x over previous
import jax
import jax.numpy as jnp
from jax.experimental import pallas as pl
from jax.experimental.pallas import tpu as pltpu

_LOG2E = 1.4426950408889634
_LN2 = 0.6931471805599453


def _focal_body(x_ref, t_ref, o_ref, acc_ref, cnt_ref):
    i = pl.program_id(0)
    g = pl.num_programs(0)

    @pl.when(i == 0)
    def _():
        acc_ref[...] = jnp.zeros_like(acc_ref)
        cnt_ref[0] = 0.0

    x = x_ref[...]
    t = t_ref[...]
    u = jnp.abs(x)
    e = jnp.exp2(u * (-_LOG2E))
    a = 1.0 / (1.0 + e)
    l = jnp.log2(a) * (-_LN2)
    q = e * a
    h = u + l
    p_val = (a * a) * h
    q_val = (q * q) * l
    tpos = t > 0.0
    pick_p = (x >= 0.0) != tpos
    val = jnp.where(pick_p, p_val, q_val)
    alpha = jnp.where(tpos, 0.25, 0.75)
    loss = alpha * val

    br, c = loss.shape
    acc_ref[...] += jnp.sum(loss.reshape(br // 8, 8, c), axis=0)
    cnt_ref[0] += jnp.sum(jnp.max(t, axis=1))

    @pl.when(i == g - 1)
    def _():
        o_ref[0, 0] = jnp.sum(acc_ref[...]) / jnp.maximum(cnt_ref[0], 1.0)


def kernel(logits, targets):
    n, c = logits.shape
    br = 5376
    grid = n // br
    out = pl.pallas_call(
        _focal_body,
        grid=(grid,),
        in_specs=[
            pl.BlockSpec((br, c), lambda i: (i, 0)),
            pl.BlockSpec((br, c), lambda i: (i, 0)),
        ],
        out_specs=pl.BlockSpec((1, 1), lambda i: (0, 0), memory_space=pltpu.SMEM),
        out_shape=jax.ShapeDtypeStruct((1, 1), jnp.float32),
        scratch_shapes=[
            pltpu.VMEM((8, c), jnp.float32),
            pltpu.SMEM((2,), jnp.float32),
        ],
        compiler_params=pltpu.CompilerParams(
            dimension_semantics=("arbitrary",),
        ),
    )(logits, targets)
    return out[0, 0]
